# Initial kernel scaffold; baseline (speedup 1.0000x reference)
import jax
import jax.numpy as jnp
from jax.experimental import pallas as pl


def kernel(x, rel_pos_emb):
    raise NotImplementedError("write your pallas kernel here")


# speedup vs baseline: 6.3927x; 6.3927x over previous
import jax
import jax.numpy as jnp
from jax import lax
from jax.experimental import pallas as pl
from jax.experimental.pallas import tpu as pltpu
from jax.experimental.pallas import tpu_sc as plsc

MAXP = 2048
SEQ = 2048
D = 64
T = 2 * MAXP + 1
SCALE = D ** (-0.5)

NC, NS = 2, 16
NW = NC * NS
NBJ = 2
NBI = NW // NBJ
BI = SEQ // NBI
BJ = SEQ // NBJ
W = BI + BJ - 1
WE = W * D
RE = BJ * D
NBUF = 4


def _scale_body(t_ref, o_ref):
    o_ref[...] = t_ref[...] * SCALE


def _scale_table(tab):
    return pl.pallas_call(
        _scale_body,
        out_shape=jax.ShapeDtypeStruct((T, D), jnp.float32),
    )(tab)


def _sc_body(tab_hbm, out_hbm, win, s0, s1, s2, s3):
    sems = (s0, s1, s2, s3)
    c = lax.axis_index("c")
    s = lax.axis_index("s")
    wid = s * NC + c
    bi = wid // NBJ
    bj = wid - bi * NBJ
    i0 = bi * BI
    j0 = bj * BJ
    lo = MAXP + j0 - i0 - (BI - 1)
    pltpu.sync_copy(tab_hbm.at[pl.ds(lo * D, WE)], win)

    def cbody(t, carry):
        handles = []
        for b in range(NBUF):
            r = t * NBUF + b
            src = win.at[pl.ds((BI - 1 - r) * D, RE)]
            dst = out_hbm.at[pl.ds(((i0 + r) * SEQ + j0) * D, RE)]
            handles.append(pltpu.async_copy(src, dst, sems[b]))
        for h in handles:
            h.wait()
        return carry

    lax.fori_loop(0, BI // NBUF, cbody, 0)


def _materialize(flat_tab):
    mesh = plsc.VectorSubcoreMesh(
        core_axis_name="c", subcore_axis_name="s",
        num_cores=NC, num_subcores=NS,
    )
    f = pl.kernel(
        _sc_body,
        out_type=jax.ShapeDtypeStruct((SEQ * SEQ * D,), jnp.float32),
        mesh=mesh,
        scratch_types=[
            pltpu.VMEM((WE,), jnp.float32),
            pltpu.SemaphoreType.DMA,
            pltpu.SemaphoreType.DMA,
            pltpu.SemaphoreType.DMA,
            pltpu.SemaphoreType.DMA,
        ],
    )
    return f(flat_tab)


@jax.jit
def kernel(x, rel_pos_emb):
    del x
    scaled = _scale_table(rel_pos_emb)
    out = _materialize(scaled.reshape(-1))
    return out.reshape(SEQ, SEQ, D)


# speedup vs baseline: 53.0226x; 8.2942x over previous
import jax
import jax.numpy as jnp
from jax import lax
from jax.experimental import pallas as pl
from jax.experimental.pallas import tpu as pltpu

MAXP = 2048
SEQ = 2048
D = 64
T = 2 * MAXP + 1
SCALE = D ** (-0.5)
PAD = 4352
NB = 128
M = SEQ // NB


def _tc_body(tab_ref, o_ref, st, sem):
    b = pl.program_id(0)
    slot = lax.rem(b, 2)
    shift = lax.rem(PAD - b, PAD)
    rolled = pltpu.roll(tab_ref[...], shift, axis=1) * SCALE

    def drain(s):
        for _ in range(M):
            pltpu.make_async_copy(
                st.at[0, :, pl.ds(0, SEQ)], o_ref.at[0], s
            ).wait()

    @pl.when(b >= 2)
    def _():
        drain(sem.at[slot])

    st[slot] = rolled

    is0 = jnp.where(b == 0, 1, 0)
    for m in range(M):
        a = m + is0
        i = MAXP - (128 * a + b)
        pltpu.make_async_copy(
            st.at[slot, :, pl.ds(a * 128, SEQ)], o_ref.at[i], sem.at[slot]
        ).start()

    @pl.when(b == NB - 1)
    def _():
        drain(sem.at[1 - slot])
        drain(sem.at[slot])


def _materialize(tab_pad):
    return pl.pallas_call(
        _tc_body,
        grid=(NB,),
        in_specs=[pl.BlockSpec((D, PAD), lambda b: (0, 0))],
        out_specs=pl.BlockSpec(memory_space=pl.ANY),
        out_shape=jax.ShapeDtypeStruct((SEQ, D, SEQ), jnp.float32),
        scratch_shapes=[
            pltpu.VMEM((2, D, PAD), jnp.float32),
            pltpu.SemaphoreType.DMA((2,)),
        ],
    )(tab_pad)


@jax.jit
def kernel(x, rel_pos_emb):
    del x
    tab_pad = jnp.pad(rel_pos_emb.T, ((0, 0), (0, PAD - T)))
    out_phys = _materialize(tab_pad)
    return out_phys.transpose(0, 2, 1)


# speedup vs baseline: 53.0409x; 1.0003x over previous
import jax
import jax.numpy as jnp
from jax import lax
from jax.experimental import pallas as pl
from jax.experimental.pallas import tpu as pltpu

MAXP = 2048
SEQ = 2048
D = 64
T = 2 * MAXP + 1
SCALE = D ** (-0.5)
PAD = 4352
NB = 128
M = SEQ // NB


def _tc_body(tab_ref, o_ref, st, sem):
    b = pl.program_id(0)
    slot = lax.rem(b, 2)
    shift = lax.rem(PAD - b, PAD)
    rolled = pltpu.roll(tab_ref[...], shift, axis=1) * SCALE

    def drain(s):
        pltpu.make_async_copy(
            o_ref.at[pl.ds(0, M)], o_ref.at[pl.ds(0, M)], s
        ).wait()

    @pl.when(b >= 2)
    def _():
        drain(sem.at[slot])

    st[slot] = rolled

    is0 = jnp.where(b == 0, 1, 0)
    for m in range(M):
        a = m + is0
        i = MAXP - (128 * a + b)
        pltpu.make_async_copy(
            st.at[slot, :, pl.ds(a * 128, SEQ)], o_ref.at[i], sem.at[slot]
        ).start()

    @pl.when(b == NB - 1)
    def _():
        drain(sem.at[1 - slot])
        drain(sem.at[slot])


def _materialize(tab_pad):
    return pl.pallas_call(
        _tc_body,
        grid=(NB,),
        in_specs=[pl.BlockSpec((D, PAD), lambda b: (0, 0))],
        out_specs=pl.BlockSpec(memory_space=pl.ANY),
        out_shape=jax.ShapeDtypeStruct((SEQ, D, SEQ), jnp.float32),
        scratch_shapes=[
            pltpu.VMEM((2, D, PAD), jnp.float32),
            pltpu.SemaphoreType.DMA((2,)),
        ],
    )(tab_pad)


@jax.jit
def kernel(x, rel_pos_emb):
    del x
    tab_pad = jnp.pad(rel_pos_emb.T, ((0, 0), (0, PAD - T)))
    out_phys = _materialize(tab_pad)
    return out_phys.transpose(0, 2, 1)
